# baseline (speedup 1.0000x reference)
import jax
import jax.numpy as jnp
from jax.experimental import pallas as pl
from jax.experimental.pallas import tpu as pltpu


def _gcn_kernel(t_ref, x_ref, f_ref, o_ref, xf_ref):
    @pl.when(pl.program_id(0) == 0)
    def _compute_xf():
        xf_ref[...] = jnp.dot(
            x_ref[...], f_ref[...], preferred_element_type=jnp.float32
        )

    o_ref[...] = jnp.dot(
        t_ref[...].astype(jnp.bfloat16),
        xf_ref[...].astype(jnp.bfloat16),
        preferred_element_type=jnp.float32,
    )


def kernel(x, transform, filters):
    n, n_feat = x.shape
    n_filt = filters.shape[1]

    bm = 200
    grid = (n // bm,)

    return pl.pallas_call(
        _gcn_kernel,
        grid=grid,
        in_specs=[
            pl.BlockSpec((bm, n), lambda m: (m, 0)),
            pl.BlockSpec((n, n_feat), lambda m: (0, 0)),
            pl.BlockSpec((n_feat, n_filt), lambda m: (0, 0)),
        ],
        out_specs=pl.BlockSpec((bm, n_filt), lambda m: (m, 0)),
        out_shape=jax.ShapeDtypeStruct((n, n_filt), jnp.float32),
        scratch_shapes=[pltpu.VMEM((n, n_filt), jnp.float32)],
        compiler_params=pltpu.CompilerParams(
            dimension_semantics=("arbitrary",),
        ),
    )(transform, x, filters)


# speedup vs baseline: 1.0061x; 1.0061x over previous
import jax
import jax.numpy as jnp
from jax.experimental import pallas as pl
from jax.experimental.pallas import tpu as pltpu


def _gcn_kernel(t_ref, x_ref, f_ref, o_ref, xf_ref):
    @pl.when(pl.program_id(0) == 0)
    def _compute_xf():
        xf_ref[...] = jnp.dot(
            x_ref[...], f_ref[...], preferred_element_type=jnp.float32
        )

    o_ref[...] = jnp.dot(
        t_ref[...].astype(jnp.bfloat16),
        xf_ref[...].astype(jnp.bfloat16),
        preferred_element_type=jnp.float32,
    )


def kernel(x, transform, filters):
    n, n_feat = x.shape
    n_filt = filters.shape[1]

    bm = 400
    grid = (n // bm,)

    return pl.pallas_call(
        _gcn_kernel,
        grid=grid,
        in_specs=[
            pl.BlockSpec((bm, n), lambda m: (m, 0)),
            pl.BlockSpec((n, n_feat), lambda m: (0, 0)),
            pl.BlockSpec((n_feat, n_filt), lambda m: (0, 0)),
        ],
        out_specs=pl.BlockSpec((bm, n_filt), lambda m: (m, 0)),
        out_shape=jax.ShapeDtypeStruct((n, n_filt), jnp.float32),
        scratch_shapes=[pltpu.VMEM((n, n_filt), jnp.float32)],
        compiler_params=pltpu.CompilerParams(
            dimension_semantics=("arbitrary",),
        ),
    )(transform, x, filters)
